# Initial kernel scaffold; baseline (speedup 1.0000x reference)
import jax
import jax.numpy as jnp
from jax.experimental import pallas as pl


def kernel(x, edge_index, W1, b1, W2, b2):
    raise NotImplementedError("write your pallas kernel here")


# speedup vs baseline: 20.9854x; 20.9854x over previous
import functools

import jax
import jax.numpy as jnp
from jax import lax
from jax.experimental import pallas as pl
from jax.experimental.pallas import tpu as pltpu
from jax.experimental.pallas import tpu_sc as plsc

_N = 10000
_E = 320000
_D = 128
_DH = _D // 2
_NP = 10240
_NC = 2
_NS = 16
_NW = _NC * _NS
_EPT = _E // _NW
_K = 100
_NCHUNK = _EPT // _K
_NPAIR = _NCHUNK // 2
_STRIPE = _NP // _NS

_mesh = plsc.VectorSubcoreMesh(core_axis_name="c", subcore_axis_name="s")


@functools.partial(
    pl.kernel,
    out_type=jax.ShapeDtypeStruct((_NC, _NP), jnp.float32),
    mesh=_mesh,
    compiler_params=pltpu.CompilerParams(use_tc_tiling_on_sc=False),
    scratch_types=[
        pltpu.VMEM((_NCHUNK, _K), jnp.int32),
        pltpu.VMEM((_K,), jnp.float32),
        pltpu.VMEM_SHARED((_NP,), jnp.float32),
    ],
)
def _deg_sc(dst_hbm, ones_hbm, z_hbm, out_hbm, didx, ones_v, acc):
    cid = lax.axis_index("c")
    sid = lax.axis_index("s")
    wid = sid * _NC + cid
    r0 = sid * _STRIPE
    pltpu.sync_copy(z_hbm, acc.at[pl.ds(r0, _STRIPE)])
    pltpu.sync_copy(ones_hbm, ones_v)
    pltpu.sync_copy(dst_hbm.at[wid], didx)
    plsc.subcore_barrier()

    def body(j, carry):
        pltpu.sync_copy(ones_v, acc.at[didx.at[j]], add=True)
        return carry

    lax.fori_loop(0, _NCHUNK, body, 0)
    plsc.subcore_barrier()
    pltpu.sync_copy(acc.at[pl.ds(r0, _STRIPE)],
                    out_hbm.at[cid].at[pl.ds(r0, _STRIPE)])


@functools.partial(
    pl.kernel,
    out_type=jax.ShapeDtypeStruct((_NC, _NP, _DH), jnp.float32),
    mesh=_mesh,
    compiler_params=pltpu.CompilerParams(use_tc_tiling_on_sc=False),
    scratch_types=[
        pltpu.VMEM((_NCHUNK, _K), jnp.int32),
        pltpu.VMEM((_NCHUNK, _K), jnp.int32),
        pltpu.VMEM((_K, _DH), jnp.float32),
        pltpu.VMEM((_K, _DH), jnp.float32),
        pltpu.VMEM_SHARED((_NP, _DH), jnp.float32),
        pltpu.SemaphoreType.DMA,
        pltpu.SemaphoreType.DMA,
    ],
)
def _agg_sc(g_hbm, src_hbm, dst_hbm, z_hbm, out_hbm,
            sidx, didx, bufa, bufb, acc, sema, semb):
    cid = lax.axis_index("c")
    sid = lax.axis_index("s")
    wid = sid * _NC + cid
    r0 = sid * _STRIPE
    pltpu.sync_copy(z_hbm, acc.at[pl.ds(r0, _STRIPE)])
    pltpu.sync_copy(src_hbm.at[wid], sidx)
    pltpu.sync_copy(dst_hbm.at[wid], didx)
    plsc.subcore_barrier()

    pltpu.async_copy(g_hbm.at[sidx.at[0]], bufa, sema)

    def body(p, carry):
        j = 2 * p
        pltpu.async_copy(g_hbm.at[sidx.at[j + 1]], bufb, semb)
        pltpu.make_async_copy(g_hbm.at[sidx.at[j]], bufa, sema).wait()
        pltpu.sync_copy(bufa, acc.at[didx.at[j]], add=True)

        @pl.when(p < _NPAIR - 1)
        def _():
            pltpu.async_copy(g_hbm.at[sidx.at[j + 2]], bufa, sema)

        pltpu.make_async_copy(g_hbm.at[sidx.at[j + 1]], bufb, semb).wait()
        pltpu.sync_copy(bufb, acc.at[didx.at[j + 1]], add=True)
        return carry

    lax.fori_loop(0, _NPAIR, body, 0)
    plsc.subcore_barrier()
    pltpu.sync_copy(acc.at[pl.ds(r0, _STRIPE)],
                    out_hbm.at[cid].at[pl.ds(r0, _STRIPE)])


_RB = 1024


def _mm1_body(x_ref, w_ref, p0_ref, p1_ref, glo_ref, ghi_ref):
    dinv = lax.rsqrt(p0_ref[...] + p1_ref[...] + 1.0)
    h = jnp.dot(x_ref[...], w_ref[...], preferred_element_type=jnp.float32)
    g = h * dinv
    glo_ref[...] = g[:, :_DH]
    ghi_ref[...] = g[:, _DH:]


def _mm2_body(pl0_ref, pl1_ref, ph0_ref, ph1_ref, glo_ref, ghi_ref,
              p0_ref, p1_ref, b_ref, w_ref, olo_ref, ohi_ref):
    dinv = lax.rsqrt(p0_ref[...] + p1_ref[...] + 1.0)
    agg_lo = pl0_ref[...] + pl1_ref[...] + glo_ref[...]
    agg_hi = ph0_ref[...] + ph1_ref[...] + ghi_ref[...]
    agg = jnp.concatenate([agg_lo, agg_hi], axis=1)
    x2 = jnp.maximum(dinv * agg + b_ref[...], 0.0)
    g = jnp.dot(x2, w_ref[...], preferred_element_type=jnp.float32) * dinv
    olo_ref[...] = g[:, :_DH]
    ohi_ref[...] = g[:, _DH:]


def _out_body(pl0_ref, pl1_ref, ph0_ref, ph1_ref, glo_ref, ghi_ref,
              p0_ref, p1_ref, b_ref, o_ref):
    dinv = lax.rsqrt(p0_ref[...] + p1_ref[...] + 1.0)
    agg_lo = pl0_ref[...] + pl1_ref[...] + glo_ref[...]
    agg_hi = ph0_ref[...] + ph1_ref[...] + ghi_ref[...]
    agg = jnp.concatenate([agg_lo, agg_hi], axis=1)
    o_ref[...] = dinv * agg + b_ref[...]


def _half_spec():
    return pl.BlockSpec((_RB, _DH), lambda i: (i, 0))


def _full_spec():
    return pl.BlockSpec((_RB, _D), lambda i: (i, 0))


def _col_spec():
    return pl.BlockSpec((_RB, 1), lambda i: (i, 0))


_half_out = jax.ShapeDtypeStruct((_NP, _DH), jnp.float32)


def _mm1(xp, w, p0, p1):
    return pl.pallas_call(
        _mm1_body,
        grid=(_NP // _RB,),
        in_specs=[
            _full_spec(),
            pl.BlockSpec((_D, _D), lambda i: (0, 0)),
            _col_spec(),
            _col_spec(),
        ],
        out_specs=[_half_spec(), _half_spec()],
        out_shape=[_half_out, _half_out],
    )(xp, w, p0, p1)


def _mm2(plo0, plo1, phi0, phi1, glo, ghi, p0, p1, b, w):
    return pl.pallas_call(
        _mm2_body,
        grid=(_NP // _RB,),
        in_specs=[
            _half_spec(), _half_spec(), _half_spec(), _half_spec(),
            _half_spec(), _half_spec(),
            _col_spec(), _col_spec(),
            pl.BlockSpec((1, _D), lambda i: (0, 0)),
            pl.BlockSpec((_D, _D), lambda i: (0, 0)),
        ],
        out_specs=[_half_spec(), _half_spec()],
        out_shape=[_half_out, _half_out],
    )(plo0, plo1, phi0, phi1, glo, ghi, p0, p1, b, w)


def _out_tc(plo0, plo1, phi0, phi1, glo, ghi, p0, p1, b):
    return pl.pallas_call(
        _out_body,
        grid=(_NP // _RB,),
        in_specs=[
            _half_spec(), _half_spec(), _half_spec(), _half_spec(),
            _half_spec(), _half_spec(),
            _col_spec(), _col_spec(),
            pl.BlockSpec((1, _D), lambda i: (0, 0)),
        ],
        out_specs=_full_spec(),
        out_shape=jax.ShapeDtypeStruct((_NP, _D), jnp.float32),
    )(plo0, plo1, phi0, phi1, glo, ghi, p0, p1, b)


def kernel(x, edge_index, W1, b1, W2, b2):
    src = edge_index[0].astype(jnp.int32).reshape(_NW, _NCHUNK, _K)
    dst = edge_index[1].astype(jnp.int32).reshape(_NW, _NCHUNK, _K)
    xp = jnp.zeros((_NP, _D), jnp.float32).at[:_N].set(x)
    z_rows = jnp.zeros((_STRIPE, _DH), jnp.float32)
    z_col = jnp.zeros((_STRIPE,), jnp.float32)
    ones_k = jnp.ones((_K,), jnp.float32)

    pdeg = _deg_sc(dst, ones_k, z_col)
    p0 = pdeg[0].reshape(_NP, 1)
    p1 = pdeg[1].reshape(_NP, 1)

    glo, ghi = _mm1(xp, W1, p0, p1)
    palo = _agg_sc(glo, src, dst, z_rows)
    pahi = _agg_sc(ghi, src, dst, z_rows)
    glo2, ghi2 = _mm2(palo[0], palo[1], pahi[0], pahi[1], glo, ghi,
                      p0, p1, b1.reshape(1, _D), W2)
    pblo = _agg_sc(glo2, src, dst, z_rows)
    pbhi = _agg_sc(ghi2, src, dst, z_rows)
    outp = _out_tc(pblo[0], pblo[1], pbhi[0], pbhi[1], glo2, ghi2,
                   p0, p1, b2.reshape(1, _D))
    return outp[:_N]


# speedup vs baseline: 26.0910x; 1.2433x over previous
import functools

import jax
import jax.numpy as jnp
from jax import lax
from jax.experimental import pallas as pl
from jax.experimental.pallas import tpu as pltpu
from jax.experimental.pallas import tpu_sc as plsc

_N = 10000
_E = 320000
_D = 128
_DH = _D // 2
_NP = 10240
_NC = 2
_NS = 16
_NW = _NC * _NS
_EPT = _E // _NW
_K = 100
_NCHUNK = _EPT // _K
_NPAIR = _NCHUNK // 2
_STRIPE = _NP // _NS

_mesh = plsc.VectorSubcoreMesh(core_axis_name="c", subcore_axis_name="s")


@functools.partial(
    pl.kernel,
    out_type=jax.ShapeDtypeStruct((_NC, _NP), jnp.float32),
    mesh=_mesh,
    compiler_params=pltpu.CompilerParams(use_tc_tiling_on_sc=False),
    scratch_types=[
        pltpu.VMEM((_NCHUNK, _K), jnp.int32),
        pltpu.VMEM((_K,), jnp.float32),
        pltpu.VMEM_SHARED((_NP,), jnp.float32),
    ],
)
def _deg_sc(dst_hbm, ones_hbm, z_hbm, out_hbm, didx, ones_v, acc):
    cid = lax.axis_index("c")
    sid = lax.axis_index("s")
    wid = sid * _NC + cid
    r0 = sid * _STRIPE
    pltpu.sync_copy(z_hbm, acc.at[pl.ds(r0, _STRIPE)])
    pltpu.sync_copy(ones_hbm, ones_v)
    pltpu.sync_copy(dst_hbm.at[wid], didx)
    plsc.subcore_barrier()

    def body(j, carry):
        pltpu.sync_copy(ones_v, acc.at[didx.at[j]], add=True)
        return carry

    lax.fori_loop(0, _NCHUNK, body, 0)
    plsc.subcore_barrier()
    pltpu.sync_copy(acc.at[pl.ds(r0, _STRIPE)],
                    out_hbm.at[cid].at[pl.ds(r0, _STRIPE)])


_NBUF = 4
_NQ = _NCHUNK // _NBUF


@functools.partial(
    pl.kernel,
    out_type=jax.ShapeDtypeStruct((2, _NC, _NP, _DH), jnp.float32),
    mesh=_mesh,
    compiler_params=pltpu.CompilerParams(use_tc_tiling_on_sc=False),
    scratch_types=[
        pltpu.VMEM((_NCHUNK, _K), jnp.int32),
        pltpu.VMEM((_NCHUNK, _K), jnp.int32),
        [pltpu.VMEM((_K, _DH), jnp.float32)] * _NBUF,
        pltpu.VMEM_SHARED((_NP, _DH), jnp.float32),
        [pltpu.SemaphoreType.DMA] * _NBUF,
    ],
)
def _agg_sc(glo_hbm, ghi_hbm, src_hbm, dst_hbm, z_hbm, out_hbm,
            sidx, didx, bufs, acc, gsems):
    cid = lax.axis_index("c")
    sid = lax.axis_index("s")
    wid = sid * _NC + cid
    r0 = sid * _STRIPE
    pltpu.sync_copy(src_hbm.at[wid], sidx)
    pltpu.sync_copy(dst_hbm.at[wid], didx)

    for h, g_hbm in ((0, glo_hbm), (1, ghi_hbm)):
        for b in range(_NBUF):
            pltpu.async_copy(g_hbm.at[sidx.at[b]], bufs[b], gsems[b])
        pltpu.sync_copy(z_hbm, acc.at[pl.ds(r0, _STRIPE)])
        plsc.subcore_barrier()

        def body(p, carry):
            for b in range(_NBUF):
                j = _NBUF * p + b
                pltpu.make_async_copy(g_hbm.at[sidx.at[j]], bufs[b],
                                      gsems[b]).wait()
                pltpu.sync_copy(bufs[b], acc.at[didx.at[j]], add=True)

                @pl.when(j + _NBUF < _NCHUNK)
                def _():
                    pltpu.async_copy(g_hbm.at[sidx.at[j + _NBUF]], bufs[b],
                                     gsems[b])

            return carry

        lax.fori_loop(0, _NQ, body, 0)
        plsc.subcore_barrier()
        pltpu.sync_copy(acc.at[pl.ds(r0, _STRIPE)],
                        out_hbm.at[h].at[cid].at[pl.ds(r0, _STRIPE)])
        if h == 0:
            plsc.subcore_barrier()


_RB = 1024


def _mm1_body(x_ref, w_ref, p0_ref, p1_ref, glo_ref, ghi_ref):
    dinv = lax.rsqrt(p0_ref[...] + p1_ref[...] + 1.0)
    h = jnp.dot(x_ref[...], w_ref[...], preferred_element_type=jnp.float32)
    g = h * dinv
    glo_ref[...] = g[:, :_DH]
    ghi_ref[...] = g[:, _DH:]


def _mm2_body(pl0_ref, pl1_ref, ph0_ref, ph1_ref, glo_ref, ghi_ref,
              p0_ref, p1_ref, b_ref, w_ref, olo_ref, ohi_ref):
    dinv = lax.rsqrt(p0_ref[...] + p1_ref[...] + 1.0)
    agg_lo = pl0_ref[...] + pl1_ref[...] + glo_ref[...]
    agg_hi = ph0_ref[...] + ph1_ref[...] + ghi_ref[...]
    agg = jnp.concatenate([agg_lo, agg_hi], axis=1)
    x2 = jnp.maximum(dinv * agg + b_ref[...], 0.0)
    g = jnp.dot(x2, w_ref[...], preferred_element_type=jnp.float32) * dinv
    olo_ref[...] = g[:, :_DH]
    ohi_ref[...] = g[:, _DH:]


def _out_body(pl0_ref, pl1_ref, ph0_ref, ph1_ref, glo_ref, ghi_ref,
              p0_ref, p1_ref, b_ref, o_ref):
    dinv = lax.rsqrt(p0_ref[...] + p1_ref[...] + 1.0)
    agg_lo = pl0_ref[...] + pl1_ref[...] + glo_ref[...]
    agg_hi = ph0_ref[...] + ph1_ref[...] + ghi_ref[...]
    agg = jnp.concatenate([agg_lo, agg_hi], axis=1)
    o_ref[...] = dinv * agg + b_ref[...]


def _half_spec():
    return pl.BlockSpec((_RB, _DH), lambda i: (i, 0))


def _full_spec():
    return pl.BlockSpec((_RB, _D), lambda i: (i, 0))


def _col_spec():
    return pl.BlockSpec((_RB, 1), lambda i: (i, 0))


_half_out = jax.ShapeDtypeStruct((_NP, _DH), jnp.float32)


def _mm1(xp, w, p0, p1):
    return pl.pallas_call(
        _mm1_body,
        grid=(_NP // _RB,),
        in_specs=[
            _full_spec(),
            pl.BlockSpec((_D, _D), lambda i: (0, 0)),
            _col_spec(),
            _col_spec(),
        ],
        out_specs=[_half_spec(), _half_spec()],
        out_shape=[_half_out, _half_out],
    )(xp, w, p0, p1)


def _mm2(plo0, plo1, phi0, phi1, glo, ghi, p0, p1, b, w):
    return pl.pallas_call(
        _mm2_body,
        grid=(_NP // _RB,),
        in_specs=[
            _half_spec(), _half_spec(), _half_spec(), _half_spec(),
            _half_spec(), _half_spec(),
            _col_spec(), _col_spec(),
            pl.BlockSpec((1, _D), lambda i: (0, 0)),
            pl.BlockSpec((_D, _D), lambda i: (0, 0)),
        ],
        out_specs=[_half_spec(), _half_spec()],
        out_shape=[_half_out, _half_out],
    )(plo0, plo1, phi0, phi1, glo, ghi, p0, p1, b, w)


def _out_tc(plo0, plo1, phi0, phi1, glo, ghi, p0, p1, b):
    return pl.pallas_call(
        _out_body,
        grid=(_NP // _RB,),
        in_specs=[
            _half_spec(), _half_spec(), _half_spec(), _half_spec(),
            _half_spec(), _half_spec(),
            _col_spec(), _col_spec(),
            pl.BlockSpec((1, _D), lambda i: (0, 0)),
        ],
        out_specs=_full_spec(),
        out_shape=jax.ShapeDtypeStruct((_NP, _D), jnp.float32),
    )(plo0, plo1, phi0, phi1, glo, ghi, p0, p1, b)


def kernel(x, edge_index, W1, b1, W2, b2):
    src = edge_index[0].astype(jnp.int32).reshape(_NW, _NCHUNK, _K)
    dst = edge_index[1].astype(jnp.int32).reshape(_NW, _NCHUNK, _K)
    xp = jnp.zeros((_NP, _D), jnp.float32).at[:_N].set(x)
    z_rows = jnp.zeros((_STRIPE, _DH), jnp.float32)
    z_col = jnp.zeros((_STRIPE,), jnp.float32)
    ones_k = jnp.ones((_K,), jnp.float32)

    pdeg = _deg_sc(dst, ones_k, z_col)
    p0 = pdeg[0].reshape(_NP, 1)
    p1 = pdeg[1].reshape(_NP, 1)

    glo, ghi = _mm1(xp, W1, p0, p1)
    pa = _agg_sc(glo, ghi, src, dst, z_rows)
    glo2, ghi2 = _mm2(pa[0, 0], pa[0, 1], pa[1, 0], pa[1, 1], glo, ghi,
                      p0, p1, b1.reshape(1, _D), W2)
    pb = _agg_sc(glo2, ghi2, src, dst, z_rows)
    outp = _out_tc(pb[0, 0], pb[0, 1], pb[1, 0], pb[1, 1], glo2, ghi2,
                   p0, p1, b2.reshape(1, _D))
    return outp[:_N]


# speedup vs baseline: 26.3330x; 1.0093x over previous
import functools

import jax
import jax.numpy as jnp
from jax import lax
from jax.experimental import pallas as pl
from jax.experimental.pallas import tpu as pltpu
from jax.experimental.pallas import tpu_sc as plsc

_N = 10000
_E = 320000
_D = 128
_DH = _D // 2
_NP = 10240
_NC = 2
_NS = 16
_NW = _NC * _NS
_EPT = _E // _NW
_K = 100
_NCHUNK = _EPT // _K
_STRIPE = _NP // _NS
_NBUF = 4
_NQ = _NCHUNK // _NBUF

_mesh = plsc.VectorSubcoreMesh(core_axis_name="c", subcore_axis_name="s")


@functools.partial(
    pl.kernel,
    out_type=jax.ShapeDtypeStruct((_NC, _NP), jnp.float32),
    mesh=_mesh,
    compiler_params=pltpu.CompilerParams(use_tc_tiling_on_sc=False),
    scratch_types=[
        pltpu.VMEM((_NCHUNK, _K), jnp.int32),
        pltpu.VMEM((_K,), jnp.float32),
        pltpu.VMEM_SHARED((_NP,), jnp.float32),
    ],
)
def _deg_sc(dst_hbm, ones_hbm, z_hbm, out_hbm, didx, ones_v, acc):
    cid = lax.axis_index("c")
    sid = lax.axis_index("s")
    wid = sid * _NC + cid
    r0 = sid * _STRIPE
    pltpu.sync_copy(z_hbm, acc.at[pl.ds(r0, _STRIPE)])
    pltpu.sync_copy(ones_hbm, ones_v)
    pltpu.sync_copy(dst_hbm.at[wid], didx)
    plsc.subcore_barrier()

    def body(j, carry):
        pltpu.sync_copy(ones_v, acc.at[didx.at[j]], add=True)
        return carry

    lax.fori_loop(0, _NCHUNK, body, 0)
    plsc.subcore_barrier()
    pltpu.sync_copy(acc.at[pl.ds(r0, _STRIPE)],
                    out_hbm.at[cid].at[pl.ds(r0, _STRIPE)])


@functools.partial(
    pl.kernel,
    out_type=jax.ShapeDtypeStruct((2, _NC, _NP, _DH), jnp.float32),
    mesh=_mesh,
    compiler_params=pltpu.CompilerParams(use_tc_tiling_on_sc=False),
    scratch_types=[
        pltpu.VMEM((_NCHUNK, _K), jnp.int32),
        pltpu.VMEM((_NCHUNK, _K), jnp.int32),
        [pltpu.VMEM((_K, _DH), jnp.float32)] * _NBUF,
        pltpu.VMEM_SHARED((_NP, _DH), jnp.float32),
        [pltpu.SemaphoreType.DMA] * _NBUF,
    ],
)
def _agg_sc(glo_hbm, ghi_hbm, src_hbm, dst_hbm, z_hbm, out_hbm,
            sidx, didx, bufs, acc, gsems):
    cid = lax.axis_index("c")
    sid = lax.axis_index("s")
    wid = sid * _NC + cid
    r0 = sid * _STRIPE
    pltpu.sync_copy(src_hbm.at[wid], sidx)
    pltpu.sync_copy(dst_hbm.at[wid], didx)

    for h, g_hbm in ((0, glo_hbm), (1, ghi_hbm)):
        for b in range(_NBUF):
            pltpu.async_copy(g_hbm.at[sidx.at[b]], bufs[b], gsems[b])
        pltpu.sync_copy(z_hbm, acc.at[pl.ds(r0, _STRIPE)])
        plsc.subcore_barrier()

        def body(p, carry):
            for b in range(_NBUF):
                j = _NBUF * p + b
                pltpu.make_async_copy(g_hbm.at[sidx.at[j]], bufs[b],
                                      gsems[b]).wait()
                pltpu.sync_copy(bufs[b], acc.at[didx.at[j]], add=True)

                @pl.when(j + _NBUF < _NCHUNK)
                def _():
                    pltpu.async_copy(g_hbm.at[sidx.at[j + _NBUF]], bufs[b],
                                     gsems[b])

            return carry

        lax.fori_loop(0, _NQ, body, 0)
        plsc.subcore_barrier()
        pltpu.sync_copy(acc.at[pl.ds(r0, _STRIPE)],
                        out_hbm.at[h].at[cid].at[pl.ds(r0, _STRIPE)])
        if h == 0:
            plsc.subcore_barrier()


_RB = 1000


def _mm_plain_body(x_ref, w_ref, h_ref):
    h_ref[...] = jnp.dot(x_ref[...], w_ref[...],
                         preferred_element_type=jnp.float32)


def _scale_body(h_ref, p0_ref, p1_ref, glo_ref, ghi_ref):
    dinv = lax.rsqrt(p0_ref[...] + p1_ref[...] + 1.0)
    g = h_ref[...] * dinv
    glo_ref[...] = g[:, :_DH]
    ghi_ref[...] = g[:, _DH:]


def _mm2_body(pl0_ref, pl1_ref, ph0_ref, ph1_ref, glo_ref, ghi_ref,
              p0_ref, p1_ref, b_ref, w_ref, olo_ref, ohi_ref):
    dinv = lax.rsqrt(p0_ref[...] + p1_ref[...] + 1.0)
    agg_lo = pl0_ref[...] + pl1_ref[...] + glo_ref[...]
    agg_hi = ph0_ref[...] + ph1_ref[...] + ghi_ref[...]
    agg = jnp.concatenate([agg_lo, agg_hi], axis=1)
    x2 = jnp.maximum(dinv * agg + b_ref[...], 0.0)
    g = jnp.dot(x2, w_ref[...], preferred_element_type=jnp.float32) * dinv
    olo_ref[...] = g[:, :_DH]
    ohi_ref[...] = g[:, _DH:]


def _out_body(pl0_ref, pl1_ref, ph0_ref, ph1_ref, glo_ref, ghi_ref,
              p0_ref, p1_ref, b_ref, o_ref):
    dinv = lax.rsqrt(p0_ref[...] + p1_ref[...] + 1.0)
    agg_lo = pl0_ref[...] + pl1_ref[...] + glo_ref[...]
    agg_hi = ph0_ref[...] + ph1_ref[...] + ghi_ref[...]
    agg = jnp.concatenate([agg_lo, agg_hi], axis=1)
    o_ref[...] = dinv * agg + b_ref[...]


def _half_spec():
    return pl.BlockSpec((_RB, _DH), lambda i: (i, 0))


def _full_spec():
    return pl.BlockSpec((_RB, _D), lambda i: (i, 0))


def _col_spec():
    return pl.BlockSpec((_RB, 1), lambda i: (i, 0))


_half_out = jax.ShapeDtypeStruct((_N, _DH), jnp.float32)


def _mm_plain(x, w):
    return pl.pallas_call(
        _mm_plain_body,
        grid=(_N // _RB,),
        in_specs=[
            _full_spec(),
            pl.BlockSpec((_D, _D), lambda i: (0, 0)),
        ],
        out_specs=_full_spec(),
        out_shape=jax.ShapeDtypeStruct((_N, _D), jnp.float32),
    )(x, w)


def _scale(h, p0, p1):
    return pl.pallas_call(
        _scale_body,
        grid=(_N // _RB,),
        in_specs=[_full_spec(), _col_spec(), _col_spec()],
        out_specs=[_half_spec(), _half_spec()],
        out_shape=[_half_out, _half_out],
    )(h, p0, p1)


def _mm2(plo0, plo1, phi0, phi1, glo, ghi, p0, p1, b, w):
    return pl.pallas_call(
        _mm2_body,
        grid=(_N // _RB,),
        in_specs=[
            _half_spec(), _half_spec(), _half_spec(), _half_spec(),
            _half_spec(), _half_spec(),
            _col_spec(), _col_spec(),
            pl.BlockSpec((1, _D), lambda i: (0, 0)),
            pl.BlockSpec((_D, _D), lambda i: (0, 0)),
        ],
        out_specs=[_half_spec(), _half_spec()],
        out_shape=[_half_out, _half_out],
    )(plo0, plo1, phi0, phi1, glo, ghi, p0, p1, b, w)


def _out_tc(plo0, plo1, phi0, phi1, glo, ghi, p0, p1, b):
    return pl.pallas_call(
        _out_body,
        grid=(_N // _RB,),
        in_specs=[
            _half_spec(), _half_spec(), _half_spec(), _half_spec(),
            _half_spec(), _half_spec(),
            _col_spec(), _col_spec(),
            pl.BlockSpec((1, _D), lambda i: (0, 0)),
        ],
        out_specs=_full_spec(),
        out_shape=jax.ShapeDtypeStruct((_N, _D), jnp.float32),
    )(plo0, plo1, phi0, phi1, glo, ghi, p0, p1, b)


def kernel(x, edge_index, W1, b1, W2, b2):
    src = edge_index[0].astype(jnp.int32).reshape(_NW, _NCHUNK, _K)
    dst = edge_index[1].astype(jnp.int32).reshape(_NW, _NCHUNK, _K)
    z_rows = jnp.zeros((_STRIPE, _DH), jnp.float32)
    z_col = jnp.zeros((_STRIPE,), jnp.float32)
    ones_k = jnp.ones((_K,), jnp.float32)

    h1 = _mm_plain(x, W1)
    pdeg = _deg_sc(dst, ones_k, z_col)
    p0 = pdeg[0].reshape(_NP, 1)
    p1 = pdeg[1].reshape(_NP, 1)

    glo, ghi = _scale(h1, p0, p1)
    pa = _agg_sc(glo, ghi, src, dst, z_rows)
    glo2, ghi2 = _mm2(pa[0, 0], pa[0, 1], pa[1, 0], pa[1, 1],
                      glo, ghi, p0, p1, b1.reshape(1, _D), W2)
    pb = _agg_sc(glo2, ghi2, src, dst, z_rows)
    out = _out_tc(pb[0, 0], pb[0, 1], pb[1, 0], pb[1, 1],
                  glo2, ghi2, p0, p1, b2.reshape(1, _D))
    return out


# speedup vs baseline: 32.1763x; 1.2219x over previous
import functools

import jax
import jax.numpy as jnp
from jax import lax
from jax.experimental import pallas as pl
from jax.experimental.pallas import tpu as pltpu
from jax.experimental.pallas import tpu_sc as plsc

_N = 10000
_E = 320000
_D = 128
_DH = _D // 2
_NP = 10240
_NC = 2
_NS = 16
_NW = _NC * _NS
_EPT = _E // _NW
_K = 100
_NCHUNK = _EPT // _K
_STRIPE = _NP // _NS
_NBUF = 4
_NQ = _NCHUNK // _NBUF

_mesh = plsc.VectorSubcoreMesh(core_axis_name="c", subcore_axis_name="s")


@functools.partial(
    pl.kernel,
    out_type=jax.ShapeDtypeStruct((_NC, _NP), jnp.float32),
    mesh=_mesh,
    compiler_params=pltpu.CompilerParams(use_tc_tiling_on_sc=False),
    scratch_types=[
        pltpu.VMEM((_NCHUNK, _K), jnp.int32),
        pltpu.VMEM((_K,), jnp.float32),
        pltpu.VMEM_SHARED((_NP,), jnp.float32),
    ],
)
def _deg_sc(dst_hbm, ones_hbm, z_hbm, out_hbm, didx, ones_v, acc):
    cid = lax.axis_index("c")
    sid = lax.axis_index("s")
    wid = sid * _NC + cid
    r0 = sid * _STRIPE
    pltpu.sync_copy(z_hbm, acc.at[pl.ds(r0, _STRIPE)])
    pltpu.sync_copy(ones_hbm, ones_v)
    pltpu.sync_copy(dst_hbm.at[wid], didx)
    plsc.subcore_barrier()

    def body(j, carry):
        pltpu.sync_copy(ones_v, acc.at[didx.at[j]], add=True)
        return carry

    lax.fori_loop(0, _NCHUNK, body, 0)
    plsc.subcore_barrier()
    pltpu.sync_copy(acc.at[pl.ds(r0, _STRIPE)],
                    out_hbm.at[cid].at[pl.ds(r0, _STRIPE)])


@functools.partial(
    pl.kernel,
    out_type=jax.ShapeDtypeStruct((_NC, _NP, _D), jnp.float32),
    mesh=_mesh,
    compiler_params=pltpu.CompilerParams(use_tc_tiling_on_sc=False),
    scratch_types=[
        pltpu.VMEM((_NCHUNK, _K), jnp.int32),
        pltpu.VMEM((_NCHUNK, _K), jnp.int32),
        pltpu.VMEM((_NCHUNK, _K), jnp.int32),
        [pltpu.VMEM((_K, _DH), jnp.float32)] * _NBUF,
        pltpu.VMEM_SHARED((_NP, _DH), jnp.float32),
        [pltpu.SemaphoreType.DMA] * _NBUF,
    ],
)
def _agg_sc(g2_hbm, srclo_hbm, srchi_hbm, dst_hbm, z_hbm, out_hbm,
            slo, shi, didx, bufs, acc, gsems):
    cid = lax.axis_index("c")
    sid = lax.axis_index("s")
    wid = sid * _NC + cid
    r0 = sid * _STRIPE
    pltpu.sync_copy(srclo_hbm.at[wid], slo)
    pltpu.sync_copy(srchi_hbm.at[wid], shi)
    pltpu.sync_copy(dst_hbm.at[wid], didx)

    for h, sidx in ((0, slo), (1, shi)):
        for b in range(_NBUF):
            pltpu.async_copy(g2_hbm.at[sidx.at[b]], bufs[b], gsems[b])
        pltpu.sync_copy(z_hbm, acc.at[pl.ds(r0, _STRIPE)])
        plsc.subcore_barrier()

        def body(p, carry):
            for b in range(_NBUF):
                j = _NBUF * p + b
                pltpu.make_async_copy(g2_hbm.at[sidx.at[j]], bufs[b],
                                      gsems[b]).wait()
                pltpu.sync_copy(bufs[b], acc.at[didx.at[j]], add=True)

                @pl.when(j + _NBUF < _NCHUNK)
                def _():
                    pltpu.async_copy(g2_hbm.at[sidx.at[j + _NBUF]], bufs[b],
                                     gsems[b])

            return carry

        lax.fori_loop(0, _NQ, body, 0)
        plsc.subcore_barrier()
        pltpu.sync_copy(
            acc.at[pl.ds(r0, _STRIPE)],
            out_hbm.at[cid].at[pl.ds(r0, _STRIPE)].at[:, pl.ds(h * _DH, _DH)])
        if h == 0:
            plsc.subcore_barrier()


_RB = 1000


def _mm_plain_body(x_ref, w_ref, h_ref):
    h_ref[...] = jnp.dot(x_ref[...], w_ref[...],
                         preferred_element_type=jnp.float32)


def _scale_body(h_ref, p0_ref, p1_ref, g_ref):
    dinv = lax.rsqrt(p0_ref[...] + p1_ref[...] + 1.0)
    g_ref[...] = h_ref[...] * dinv


def _mm2_body(pa0_ref, pa1_ref, g_ref, p0_ref, p1_ref, b_ref, w_ref, o_ref):
    dinv = lax.rsqrt(p0_ref[...] + p1_ref[...] + 1.0)
    agg = pa0_ref[...] + pa1_ref[...] + g_ref[...]
    x2 = jnp.maximum(dinv * agg + b_ref[...], 0.0)
    o_ref[...] = jnp.dot(x2, w_ref[...], preferred_element_type=jnp.float32) * dinv


def _out_body(pb0_ref, pb1_ref, g_ref, p0_ref, p1_ref, b_ref, o_ref):
    dinv = lax.rsqrt(p0_ref[...] + p1_ref[...] + 1.0)
    agg = pb0_ref[...] + pb1_ref[...] + g_ref[...]
    o_ref[...] = dinv * agg + b_ref[...]


def _full_spec():
    return pl.BlockSpec((_RB, _D), lambda i: (i, 0))


def _col_spec():
    return pl.BlockSpec((_RB, 1), lambda i: (i, 0))


_full_out = jax.ShapeDtypeStruct((_N, _D), jnp.float32)


def _mm_plain(x, w):
    return pl.pallas_call(
        _mm_plain_body,
        grid=(_N // _RB,),
        in_specs=[
            _full_spec(),
            pl.BlockSpec((_D, _D), lambda i: (0, 0)),
        ],
        out_specs=_full_spec(),
        out_shape=_full_out,
    )(x, w)


def _scale(h, p0, p1):
    return pl.pallas_call(
        _scale_body,
        grid=(_N // _RB,),
        in_specs=[_full_spec(), _col_spec(), _col_spec()],
        out_specs=_full_spec(),
        out_shape=_full_out,
    )(h, p0, p1)


def _mm2(pa0, pa1, g, p0, p1, b, w):
    return pl.pallas_call(
        _mm2_body,
        grid=(_N // _RB,),
        in_specs=[
            _full_spec(), _full_spec(), _full_spec(),
            _col_spec(), _col_spec(),
            pl.BlockSpec((1, _D), lambda i: (0, 0)),
            pl.BlockSpec((_D, _D), lambda i: (0, 0)),
        ],
        out_specs=_full_spec(),
        out_shape=_full_out,
    )(pa0, pa1, g, p0, p1, b, w)


def _out_tc(pb0, pb1, g, p0, p1, b):
    return pl.pallas_call(
        _out_body,
        grid=(_N // _RB,),
        in_specs=[
            _full_spec(), _full_spec(), _full_spec(),
            _col_spec(), _col_spec(),
            pl.BlockSpec((1, _D), lambda i: (0, 0)),
        ],
        out_specs=_full_spec(),
        out_shape=_full_out,
    )(pb0, pb1, g, p0, p1, b)


def kernel(x, edge_index, W1, b1, W2, b2):
    src = edge_index[0].astype(jnp.int32)
    srclo = (src * 2).reshape(_NW, _NCHUNK, _K)
    srchi = (src * 2 + 1).reshape(_NW, _NCHUNK, _K)
    dst = edge_index[1].astype(jnp.int32).reshape(_NW, _NCHUNK, _K)
    z_rows = jnp.zeros((_STRIPE, _DH), jnp.float32)
    z_col = jnp.zeros((_STRIPE,), jnp.float32)
    ones_k = jnp.ones((_K,), jnp.float32)

    h1 = _mm_plain(x, W1)
    pdeg = _deg_sc(dst, ones_k, z_col)
    p0 = pdeg[0].reshape(_NP, 1)
    p1 = pdeg[1].reshape(_NP, 1)

    g1 = _scale(h1, p0, p1)
    pa = _agg_sc(g1.reshape(2 * _N, _DH), srclo, srchi, dst, z_rows)
    g2 = _mm2(pa[0], pa[1], g1, p0, p1, b1.reshape(1, _D), W2)
    pb = _agg_sc(g2.reshape(2 * _N, _DH), srclo, srchi, dst, z_rows)
    out = _out_tc(pb[0], pb[1], g2, p0, p1, b2.reshape(1, _D))
    return out
